# baseline (speedup 1.0000x reference)
import jax
import jax.numpy as jnp
from jax.experimental import pallas as pl

B, S, D = 8, 576, 256
Q, K = 4, 1024
N = B * S
TR = 512
NT = N // TR


def _stage_body(res_ref, rn_ref, ct_ref, cn_ref, cb_ref,
                idx_ref, newres_ref, qst_ref, closs_ref):
    r = res_ref[...]
    rnorm = jnp.sum(r * r, axis=1, keepdims=True)
    del rn_ref
    ct = ct_ref[0]
    cnorm = cn_ref[0]
    dot = jax.lax.dot_general(r, ct, (((1,), (0,)), ((), ())),
                              preferred_element_type=jnp.float32)
    d = (rnorm - 2.0 * dot) + cnorm
    dmin = jnp.min(d, axis=1, keepdims=True)
    iota = jax.lax.broadcasted_iota(jnp.int32, (TR, K), 1)
    idx = jnp.min(jnp.where(d == dmin, iota, K), axis=1)
    idx_ref[...] = idx
    onehot = (iota == idx[:, None]).astype(jnp.float32)
    quant = jax.lax.dot_general(
        onehot, cb_ref[0], (((1,), (0,)), ((), ())),
        preferred_element_type=jnp.float32,
        precision=jax.lax.Precision.HIGHEST)
    closs_ref[...] = jnp.sum((quant - r) ** 2).reshape(1, 1, 1)
    qst_ref[...] = r + (quant - r)
    newres_ref[...] = r - quant


def _make_stage(q):
    return pl.pallas_call(
        _stage_body,
        grid=(NT,),
        in_specs=[
            pl.BlockSpec((TR, D), lambda i: (i, 0)),
            pl.BlockSpec((TR, 1), lambda i: (i, 0)),
            pl.BlockSpec((1, D, K), lambda i: (q, 0, 0)),
            pl.BlockSpec((1, 1, K), lambda i: (q, 0, 0)),
            pl.BlockSpec((1, K, D), lambda i: (q, 0, 0)),
        ],
        out_specs=[
            pl.BlockSpec((TR,), lambda i: (i,)),
            pl.BlockSpec((TR, D), lambda i: (i, 0)),
            pl.BlockSpec((TR, D), lambda i: (i, 0)),
            pl.BlockSpec((1, 1, 1), lambda i: (i, 0, 0)),
        ],
        out_shape=[
            jax.ShapeDtypeStruct((N,), jnp.int32),
            jax.ShapeDtypeStruct((N, D), jnp.float32),
            jax.ShapeDtypeStruct((N, D), jnp.float32),
            jax.ShapeDtypeStruct((NT, 1, 1), jnp.float32),
        ],
    )


def kernel(x, codebooks):
    ct = codebooks.transpose(0, 2, 1)
    cn = jnp.sum(codebooks**2, axis=-1)[:, None, :]

    res = x
    idxs, qsts, closses = [], [], []
    for q in range(Q):
        rn = jnp.sum(res**2, axis=-1, keepdims=True)
        idx, newres, qst, closs = _make_stage(q)(
            res.reshape(N, D), rn.reshape(N, 1), ct, cn, codebooks)
        res = newres.reshape(B, S, D)
        idxs.append(idx)
        qsts.append(qst)
        closses.append(closs)

    all_indices = jnp.stack(idxs, axis=-1).reshape(B, S, Q)
    quantized_out = (((qsts[0] + qsts[1]) + qsts[2]) + qsts[3]).reshape(B, S, D)
    commit_loss = jnp.stack([c.sum() for c in closses]) / (B * S * D)
    return (all_indices, quantized_out, commit_loss)
